# baseline (speedup 1.0000x reference)
import functools

import jax
import jax.numpy as jnp
from jax import lax
from jax.experimental import pallas as pl
from jax.experimental.pallas import tpu as pltpu
from jax.experimental.pallas import tpu_sc as plsc

_LN10 = 2.302585092994046

_ROWS = 128
_COLS = 128
_NW = 32
_RPW = _ROWS // _NW
_EPW = _RPW * _COLS
_LANES = 16


def _build():
    mesh = plsc.VectorSubcoreMesh(core_axis_name="c", subcore_axis_name="s")

    @functools.partial(
        pl.kernel,
        mesh=mesh,
        out_type=jax.ShapeDtypeStruct((_ROWS * _COLS,), jnp.float32),
        scratch_types=[
            pltpu.VMEM((_RPW, _COLS), jnp.int32),
            pltpu.VMEM((_EPW,), jnp.float32),
            pltpu.VMEM((_EPW,), jnp.float32),
            pltpu.VMEM((_EPW,), jnp.float32),
            pltpu.VMEM((_EPW,), jnp.float32),
            pltpu.SemaphoreType.DMA,
            pltpu.SemaphoreType.DMA,
            pltpu.SemaphoreType.DMA,
        ],
    )
    def k(ids_hbm, mu_hbm, ls_hbm, nz_hbm, out_hbm,
          idx_v, mu_v, ls_v, nz_v, out_v, gsem, isem, osem):
        wid = lax.axis_index("s") * 2 + lax.axis_index("c")
        rbase = wid * _RPW
        ebase = wid * _EPW
        half = _EPW // 2
        hr = _RPW // 2
        nz_copy = pltpu.async_copy(nz_hbm.at[pl.ds(ebase, _EPW)], nz_v, isem)
        gathers = []
        for h in range(2):
            pltpu.sync_copy(
                ids_hbm.at[pl.ds(rbase + h * hr, hr)],
                idx_v.at[pl.ds(h * hr, hr)])
            for r in range(h * hr, (h + 1) * hr):
                gathers.append(pltpu.async_copy(
                    mu_hbm.at[idx_v.at[r]], mu_v.at[pl.ds(r * _COLS, _COLS)], gsem))
                gathers.append(pltpu.async_copy(
                    ls_hbm.at[idx_v.at[r]], ls_v.at[pl.ds(r * _COLS, _COLS)], gsem))
        nz_copy.wait()

        def body(i, _):
            sl = pl.ds(pl.multiple_of(i * _LANES, _LANES), _LANES)
            out_v[sl] = jnp.exp((mu_v[sl] + jnp.exp(ls_v[sl]) * nz_v[sl]) * _LN10)
            return _

        out_copies = []
        for h in range(2):
            for c in gathers[h * 2 * hr:(h + 1) * 2 * hr]:
                c.wait()
            lax.fori_loop(h * half // _LANES, (h + 1) * half // _LANES,
                          body, 0, unroll=4)
            out_copies.append(pltpu.async_copy(
                out_v.at[pl.ds(h * half, half)],
                out_hbm.at[pl.ds(ebase + h * half, half)], osem))
        for c in out_copies:
            c.wait()

    return k


_sc_kernel = _build()


def kernel(batch_size, family_ids, mu, log_sigma, noise):
    ids2 = family_ids.astype(jnp.int32).reshape(_ROWS, _COLS)
    out = _sc_kernel(ids2, mu, log_sigma, noise)
    return out


# speedup vs baseline: 1.0030x; 1.0030x over previous
import functools

import jax
import jax.numpy as jnp
from jax import lax
from jax.experimental import pallas as pl
from jax.experimental.pallas import tpu as pltpu
from jax.experimental.pallas import tpu_sc as plsc

_LN10 = 2.302585092994046

_ROWS = 128
_COLS = 128
_NW = 32
_RPW = _ROWS // _NW
_EPW = _RPW * _COLS
_LANES = 16


def _build():
    mesh = plsc.VectorSubcoreMesh(core_axis_name="c", subcore_axis_name="s")

    @functools.partial(
        pl.kernel,
        mesh=mesh,
        out_type=jax.ShapeDtypeStruct((_ROWS * _COLS,), jnp.float32),
        scratch_types=[
            pltpu.VMEM((_RPW, _COLS), jnp.int32),
            pltpu.VMEM((2 * _EPW,), jnp.float32),
            pltpu.VMEM((_EPW,), jnp.float32),
            pltpu.VMEM((_EPW,), jnp.float32),
            pltpu.SemaphoreType.DMA,
            pltpu.SemaphoreType.DMA,
        ],
    )
    def k(ids_hbm, mu_hbm, ls_hbm, nz_hbm, out_hbm,
          idx_v, mls_v, nz_v, out_v, gsem, isem):
        wid = lax.axis_index("s") * 2 + lax.axis_index("c")
        rbase = wid * _RPW
        ebase = wid * _EPW
        nz_copy = pltpu.async_copy(nz_hbm.at[pl.ds(ebase, _EPW)], nz_v, isem)
        pltpu.sync_copy(ids_hbm.at[pl.ds(rbase, _RPW)], idx_v)

        def fire(r, _):
            off = pl.multiple_of(r * _COLS, _COLS)
            pltpu.async_copy(
                mu_hbm.at[idx_v.at[r]], mls_v.at[pl.ds(off, _COLS)], gsem)
            pltpu.async_copy(
                ls_hbm.at[idx_v.at[r]], mls_v.at[pl.ds(_EPW + off, _COLS)], gsem)
            return _

        lax.fori_loop(0, _RPW, fire, 0)
        pltpu.make_async_copy(
            mu_hbm.at[pl.ds(0, 2 * _EPW)], mls_v, gsem).wait()
        nz_copy.wait()

        def body(i, _):
            sl = pl.ds(pl.multiple_of(i * _LANES, _LANES), _LANES)
            sl2 = pl.ds(pl.multiple_of(_EPW + i * _LANES, _LANES), _LANES)
            out_v[sl] = jnp.exp((mls_v[sl] + jnp.exp(mls_v[sl2]) * nz_v[sl]) * _LN10)
            return _

        lax.fori_loop(0, _EPW // _LANES, body, 0, unroll=4)
        pltpu.sync_copy(out_v, out_hbm.at[pl.ds(ebase, _EPW)])

    return k


_sc_kernel = _build()


def kernel(batch_size, family_ids, mu, log_sigma, noise):
    ids2 = family_ids.astype(jnp.int32).reshape(_ROWS, _COLS)
    out = _sc_kernel(ids2, mu, log_sigma, noise)
    return out


# speedup vs baseline: 1.0072x; 1.0042x over previous
import functools

import jax
import jax.numpy as jnp
from jax import lax
from jax.experimental import pallas as pl
from jax.experimental.pallas import tpu as pltpu
from jax.experimental.pallas import tpu_sc as plsc

_LN10 = 2.302585092994046

_ROWS = 128
_COLS = 128
_NW = 32
_RPW = _ROWS // _NW
_EPW = _RPW * _COLS
_LANES = 16


def _build():
    mesh = plsc.VectorSubcoreMesh(core_axis_name="c", subcore_axis_name="s")

    @functools.partial(
        pl.kernel,
        mesh=mesh,
        out_type=jax.ShapeDtypeStruct((_ROWS * _COLS,), jnp.float32),
        scratch_types=[
            pltpu.VMEM((_RPW, _COLS), jnp.int32),
            pltpu.VMEM((_EPW,), jnp.float32),
            pltpu.VMEM((_EPW,), jnp.float32),
            pltpu.VMEM((_EPW,), jnp.float32),
            pltpu.VMEM((_EPW,), jnp.float32),
            pltpu.SemaphoreType.DMA,
            pltpu.SemaphoreType.DMA,
        ],
    )
    def k(ids_hbm, mu_hbm, ls_hbm, nz_hbm, out_hbm,
          idx_v, mu_v, ls_v, nz_v, out_v, gsem, isem):
        wid = lax.axis_index("s") * 2 + lax.axis_index("c")
        rbase = wid * _RPW
        ebase = wid * _EPW
        nz_copy = pltpu.async_copy(nz_hbm.at[pl.ds(ebase, _EPW)], nz_v, isem)
        pltpu.sync_copy(ids_hbm.at[pl.ds(rbase, _RPW)], idx_v)
        gathers = []
        for r in range(_RPW):
            gathers.append(pltpu.async_copy(
                mu_hbm.at[idx_v.at[r]], mu_v.at[pl.ds(r * _COLS, _COLS)], gsem))
            gathers.append(pltpu.async_copy(
                ls_hbm.at[idx_v.at[r]], ls_v.at[pl.ds(r * _COLS, _COLS)], gsem))
        nz_copy.wait()
        for c in gathers:
            c.wait()

        def body(i, _):
            sl = pl.ds(pl.multiple_of(i * _LANES, _LANES), _LANES)
            out_v[sl] = jnp.exp((mu_v[sl] + jnp.exp(ls_v[sl]) * nz_v[sl]) * _LN10)
            return _

        lax.fori_loop(0, _EPW // _LANES, body, 0, unroll=2)
        pltpu.sync_copy(out_v, out_hbm.at[pl.ds(ebase, _EPW)])

    return k


_sc_kernel = _build()


def kernel(batch_size, family_ids, mu, log_sigma, noise):
    ids2 = family_ids.astype(jnp.int32).reshape(_ROWS, _COLS)
    out = _sc_kernel(ids2, mu, log_sigma, noise)
    return out


# speedup vs baseline: 1.0138x; 1.0066x over previous
import functools

import jax
import jax.numpy as jnp
from jax import lax
from jax.experimental import pallas as pl
from jax.experimental.pallas import tpu as pltpu
from jax.experimental.pallas import tpu_sc as plsc

_LN10 = 2.302585092994046

_ROWS = 128
_COLS = 128
_NW = 32
_RPW = _ROWS // _NW
_EPW = _RPW * _COLS
_LANES = 16


def _build():
    mesh = plsc.VectorSubcoreMesh(core_axis_name="c", subcore_axis_name="s")

    @functools.partial(
        pl.kernel,
        mesh=mesh,
        out_type=jax.ShapeDtypeStruct((_ROWS * _COLS,), jnp.float32),
        scratch_types=[
            pltpu.VMEM((_RPW, _COLS), jnp.int32),
            pltpu.VMEM((_EPW,), jnp.float32),
            pltpu.VMEM((_EPW,), jnp.float32),
            pltpu.VMEM((_EPW,), jnp.float32),
            pltpu.VMEM((_EPW,), jnp.float32),
            pltpu.SemaphoreType.DMA,
            pltpu.SemaphoreType.DMA,
        ],
    )
    def k(ids_hbm, mu_hbm, ls_hbm, nz_hbm, out_hbm,
          idx_v, mu_v, ls_v, nz_v, out_v, gsem, isem):
        wid = lax.axis_index("s") * 2 + lax.axis_index("c")
        rbase = wid * _RPW
        ebase = wid * _EPW
        nz_copy = pltpu.async_copy(nz_hbm.at[pl.ds(ebase, _EPW)], nz_v, isem)
        pltpu.sync_copy(ids_hbm.at[pl.ds(rbase, _RPW)], idx_v)
        gathers = []
        for r in range(_RPW):
            gathers.append(pltpu.async_copy(
                mu_hbm.at[idx_v.at[r]], mu_v.at[pl.ds(r * _COLS, _COLS)], gsem))
            gathers.append(pltpu.async_copy(
                ls_hbm.at[idx_v.at[r]], ls_v.at[pl.ds(r * _COLS, _COLS)], gsem))
        nz_copy.wait()
        for c in gathers:
            c.wait()

        def body(i, _):
            sl = pl.ds(pl.multiple_of(i * _LANES, _LANES), _LANES)
            out_v[sl] = jnp.exp((mu_v[sl] + jnp.exp(ls_v[sl]) * nz_v[sl]) * _LN10)
            return _

        lax.fori_loop(0, _EPW // _LANES, body, 0, unroll=4)
        pltpu.sync_copy(out_v, out_hbm.at[pl.ds(ebase, _EPW)])

    return k


_sc_kernel = _build()


def kernel(batch_size, family_ids, mu, log_sigma, noise):
    ids2 = family_ids.astype(jnp.int32).reshape(_ROWS, _COLS)
    out = _sc_kernel(ids2, mu, log_sigma, noise)
    return out


# speedup vs baseline: 1.0265x; 1.0125x over previous
import functools

import jax
import jax.numpy as jnp
from jax import lax
from jax.experimental import pallas as pl
from jax.experimental.pallas import tpu as pltpu
from jax.experimental.pallas import tpu_sc as plsc

_LN10 = 2.302585092994046

_ROWS = 128
_COLS = 128
_NW = 32
_RPW = _ROWS // _NW
_EPW = _RPW * _COLS
_LANES = 16


def _build():
    mesh = plsc.VectorSubcoreMesh(core_axis_name="c", subcore_axis_name="s")

    @functools.partial(
        pl.kernel,
        mesh=mesh,
        out_type=jax.ShapeDtypeStruct((_ROWS * _COLS,), jnp.float32),
        scratch_types=[
            pltpu.VMEM((_RPW, _COLS), jnp.int32),
            pltpu.VMEM((4 * _EPW,), jnp.float32),
            pltpu.SemaphoreType.DMA,
            pltpu.SemaphoreType.DMA,
        ],
    )
    def k(ids_hbm, mu_hbm, ls_hbm, nz_hbm, out_hbm, idx_v, buf, gsem, isem):
        wid = lax.axis_index("s") * 2 + lax.axis_index("c")
        rbase = wid * _RPW
        ebase = wid * _EPW
        nz_copy = pltpu.async_copy(
            nz_hbm.at[pl.ds(ebase, _EPW)], buf.at[pl.ds(2 * _EPW, _EPW)], isem)
        pltpu.sync_copy(ids_hbm.at[pl.ds(rbase, _RPW)], idx_v)
        for r in range(_RPW):
            pltpu.async_copy(
                mu_hbm.at[idx_v.at[r]], buf.at[pl.ds(r * _COLS, _COLS)], gsem)
            pltpu.async_copy(
                ls_hbm.at[idx_v.at[r]], buf.at[pl.ds(_EPW + r * _COLS, _COLS)], gsem)
        nz_copy.wait()
        pltpu.make_async_copy(
            mu_hbm.at[pl.ds(0, 2 * _EPW)], buf.at[pl.ds(0, 2 * _EPW)], gsem).wait()

        def body(i, _):
            off = pl.multiple_of(i * _LANES, _LANES)
            m = buf[pl.ds(off, _LANES)]
            s = buf[pl.ds(_EPW + off, _LANES)]
            z = buf[pl.ds(2 * _EPW + off, _LANES)]
            buf[pl.ds(3 * _EPW + off, _LANES)] = jnp.exp((m + jnp.exp(s) * z) * _LN10)
            return _

        lax.fori_loop(0, _EPW // _LANES, body, 0, unroll=4)
        pltpu.sync_copy(
            buf.at[pl.ds(3 * _EPW, _EPW)], out_hbm.at[pl.ds(ebase, _EPW)])

    return k


_sc_kernel = _build()


def kernel(batch_size, family_ids, mu, log_sigma, noise):
    ids2 = family_ids.astype(jnp.int32).reshape(_ROWS, _COLS)
    out = _sc_kernel(ids2, mu, log_sigma, noise)
    return out
